# Initial kernel scaffold; baseline (speedup 1.0000x reference)
import jax
import jax.numpy as jnp
from jax.experimental import pallas as pl


def kernel(ref, query):
    raise NotImplementedError("write your pallas kernel here")


# speedup vs baseline: 17.6511x; 17.6511x over previous
import jax
import jax.numpy as jnp
from jax.experimental import pallas as pl
from jax.experimental.pallas import tpu as pltpu

K = 16
QB = 256


def _knn_block(ref_ref, q_ref, d_ref, i_ref, s_ref):
    reft = ref_ref[0]
    q = q_ref[0]
    n = reft.shape[1]

    pcsq = jnp.sum(reft * reft, axis=0, keepdims=True)
    qsq = jnp.sum(q * q, axis=1, keepdims=True)
    qp = jax.lax.dot_general(q.astype(jnp.bfloat16),
                             reft.astype(jnp.bfloat16),
                             (((1,), (0,)), ((), ())),
                             preferred_element_type=jnp.float32)
    s_ref[...] = (qsq - 2.0 * qp) + pcsq

    iota = jax.lax.broadcasted_iota(jnp.int32, (QB, n), 1)
    big = jnp.int32(n)
    for k in range(K):
        s = s_ref[...]
        m = jnp.min(s, axis=1, keepdims=True)
        idx = jnp.min(jnp.where(s == m, iota, big), axis=1,
                      keepdims=True)
        d_ref[0, :, k] = jnp.sqrt(jnp.maximum(m, 1e-12))[:, 0]
        i_ref[0, :, k] = idx[:, 0]
        s_ref[...] = jnp.where(iota == idx, jnp.inf, s)


@jax.jit
def _knn(reft, query):
    b, d, n = reft.shape
    _, q, _ = query.shape
    grid = (b, q // QB)
    dist, idx = pl.pallas_call(
        _knn_block,
        grid=grid,
        in_specs=[
            pl.BlockSpec((1, d, n), lambda bi, qi: (bi, 0, 0)),
            pl.BlockSpec((1, QB, d), lambda bi, qi: (bi, qi, 0)),
        ],
        out_specs=[
            pl.BlockSpec((1, QB, K), lambda bi, qi: (bi, qi, 0)),
            pl.BlockSpec((1, QB, K), lambda bi, qi: (bi, qi, 0)),
        ],
        out_shape=[
            jax.ShapeDtypeStruct((b, q, K), jnp.float32),
            jax.ShapeDtypeStruct((b, q, K), jnp.int32),
        ],
        scratch_shapes=[pltpu.VMEM((QB, n), jnp.float32)],
    )(reft, query)
    return dist, idx


def kernel(ref, query):
    dist, idx = _knn(jnp.swapaxes(ref, 1, 2), query)
    return dist, idx.astype(jnp.int64)


# speedup vs baseline: 44.0313x; 2.4945x over previous
import functools

import jax
import jax.numpy as jnp
from jax import lax
from jax.experimental import pallas as pl
from jax.experimental.pallas import tpu as pltpu
from jax.experimental.pallas import tpu_sc as plsc

K = 16
QB = 128
NCH = 128
NW = 32
B, N, DIM, Q = 4, 16384, 64, 2048
ROWS = B * Q
RPW = ROWS // NW
CHL = N // NCH


def _score_block(ref_ref, q_ref, s_ref, cm_ref):
    reft = ref_ref[0]
    q = q_ref[0]
    pcsq = jnp.sum(reft * reft, axis=0, keepdims=True)
    qsq = jnp.sum(q * q, axis=1, keepdims=True)
    qp = lax.dot_general(q.astype(jnp.bfloat16), reft.astype(jnp.bfloat16),
                         (((1,), (0,)), ((), ())),
                         preferred_element_type=jnp.float32)
    s = (qsq - 2.0 * qp) + pcsq
    s_ref[...] = s
    m = s
    while m.shape[1] > NCH:
        w = m.shape[1] // 2
        m = jnp.minimum(m[:, :w], m[:, w:])
    cm_ref[...] = m


@jax.jit
def _scores(reft, query):
    qpb = Q // QB
    grid = (B, qpb)
    return pl.pallas_call(
        _score_block,
        grid=grid,
        in_specs=[
            pl.BlockSpec((1, DIM, N), lambda bi, qi: (bi, 0, 0)),
            pl.BlockSpec((1, QB, DIM), lambda bi, qi: (bi, qi, 0)),
        ],
        out_specs=[
            pl.BlockSpec((QB, N), lambda bi, qi: (bi * qpb + qi, 0)),
            pl.BlockSpec((QB, NCH), lambda bi, qi: (bi * qpb + qi, 0)),
        ],
        out_shape=[
            jax.ShapeDtypeStruct((ROWS, N), jnp.float32),
            jax.ShapeDtypeStruct((ROWS, NCH), jnp.float32),
        ],
    )(reft, query)


def _merge16(rv, ri, cv, ci):
    cs, cis = plsc.sort_key_val(cv, ci)
    cr = jnp.flip(cs, 0)
    cir = jnp.flip(cis, 0)
    take = cr < rv
    nv = jnp.where(take, cr, rv)
    ni = jnp.where(take, cir, ri)
    sv, si = plsc.sort_key_val(nv, ni)
    return sv, si


def _nsqrt(x):
    xb = lax.bitcast_convert_type(x, jnp.int32)
    y = lax.bitcast_convert_type(
        (xb >> 1) + jnp.int32(0x1FBD1DF5), jnp.float32)
    y = 0.5 * (y + x / y)
    y = 0.5 * (y + x / y)
    y = 0.5 * (y + x / y)
    return y


def _sc_body(s_hbm, cm_hbm, d_hbm, i_hbm,
             rowb0, rowb1, cmb0, cmb1, selb, odb, oib,
             rs0, rs1, cs0, cs1, osem):
    wid = lax.axis_index("s") * 2 + lax.axis_index("c")
    base = wid * RPW
    rowbs = (rowb0, rowb1)
    cmbs = (cmb0, cmb1)
    rsems = (rs0, rs1)
    csems = (cs0, cs1)
    lane = lax.broadcasted_iota(jnp.int32, (16,), 0)
    inf16 = jnp.full((16,), jnp.inf, jnp.float32)
    zero16 = jnp.zeros((16,), jnp.int32)

    def row_cp(r, bf):
        return pltpu.make_async_copy(s_hbm.at[base + r], rowbs[bf],
                                     rsems[bf])

    def cm_cp(r, bf):
        return pltpu.make_async_copy(cm_hbm.at[base + r], cmbs[bf],
                                     csems[bf])

    row_cp(0, 0).start()
    cm_cp(0, 0).start()
    row_cp(1, 1).start()
    cm_cp(1, 1).start()

    def process(r, bf):
        cmr = cmbs[bf]
        rowv = rowbs[bf]
        rv, ri = inf16, zero16
        for g in range(NCH // 16):
            cv = cmr[pl.ds(g * 16, 16)]
            rv, ri = _merge16(rv, ri, cv, lane + g * 16)
        t = jnp.max(rv)
        tb = jnp.broadcast_to(t, (16,))
        nsel = jnp.int32(0)
        for g in range(NCH // 16):
            cv = cmr[pl.ds(g * 16, 16)]
            msk = cv <= tb
            plsc.store_compressed(selb.at[pl.ds(nsel, 16)], lane + g * 16,
                                  mask=msk)
            nsel = nsel + jnp.sum(msk.astype(jnp.int32))

        def chunk_body(i, carry):
            rv, ri = carry
            cid = selb[pl.ds(i, 16)][0]
            r15 = jnp.max(rv)
            r15b = jnp.broadcast_to(r15, (16,))
            for g in range(CHL // 16):
                idx = cid + lane * NCH + g * (16 * NCH)
                cv = plsc.load_gather(rowv, [idx])
                hit = jnp.any(cv < r15b)

                def do_merge(args):
                    return _merge16(*args)

                def skip(args):
                    return args[0], args[1]

                rv, ri = lax.cond(hit, do_merge, skip, (rv, ri, cv, idx))
            return rv, ri

        rv, ri = lax.fori_loop(0, nsel, chunk_body, (inf16, zero16))

        d = _nsqrt(jnp.maximum(rv, 1e-12))
        odb[pl.ds(r * 16, 16)] = d
        oib[pl.ds(r * 16, 16)] = ri

    def outer(rr, _):
        for bf in range(2):
            r = rr * 2 + bf
            row_cp(r, bf).wait()
            cm_cp(r, bf).wait()
            process(r, bf)
            nxt = r + 2

            @pl.when(nxt < RPW)
            def _():
                row_cp(nxt, bf).start()
                cm_cp(nxt, bf).start()
        return 0

    lax.fori_loop(0, RPW // 2, outer, 0)
    pltpu.make_async_copy(odb, d_hbm.at[pl.ds(base * 16, RPW * 16)],
                          osem).start()
    pltpu.make_async_copy(odb, d_hbm.at[pl.ds(base * 16, RPW * 16)],
                          osem).wait()
    pltpu.sync_copy(oib, i_hbm.at[pl.ds(base * 16, RPW * 16)])


@jax.jit
def _sc_topk(s_mat, cm_mat):
    mesh = plsc.VectorSubcoreMesh(core_axis_name="c", subcore_axis_name="s")
    f = pl.kernel(
        _sc_body,
        out_type=[
            jax.ShapeDtypeStruct((ROWS * K,), jnp.float32),
            jax.ShapeDtypeStruct((ROWS * K,), jnp.int32),
        ],
        mesh=mesh,
        compiler_params=pltpu.CompilerParams(needs_layout_passes=False),
        scratch_types=[
            pltpu.VMEM((N,), jnp.float32),
            pltpu.VMEM((N,), jnp.float32),
            pltpu.VMEM((NCH,), jnp.float32),
            pltpu.VMEM((NCH,), jnp.float32),
            pltpu.VMEM((NCH + 32,), jnp.int32),
            pltpu.VMEM((RPW * K,), jnp.float32),
            pltpu.VMEM((RPW * K,), jnp.int32),
            pltpu.SemaphoreType.DMA,
            pltpu.SemaphoreType.DMA,
            pltpu.SemaphoreType.DMA,
            pltpu.SemaphoreType.DMA,
            pltpu.SemaphoreType.DMA,
        ],
    )
    return f(s_mat, cm_mat)


def kernel(ref, query):
    s_mat, cm_mat = _scores(jnp.swapaxes(ref, 1, 2), query)
    d_flat, i_flat = _sc_topk(s_mat, cm_mat)
    dist = d_flat.reshape(B, Q, K)
    idx = i_flat.reshape(B, Q, K)
    return dist, idx.astype(jnp.int64)
